# Initial kernel scaffold; baseline (speedup 1.0000x reference)
import jax
import jax.numpy as jnp
from jax.experimental import pallas as pl


def kernel(x, tables):
    raise NotImplementedError("write your pallas kernel here")


# speedup vs baseline: 1.1141x; 1.1141x over previous
import jax
import jax.numpy as jnp
from jax import lax
from jax.experimental import pallas as pl
from jax.experimental.pallas import tpu as pltpu
from jax.experimental.pallas import tpu_sc as plsc

NUM_CORES = 2
NUM_SUBCORES = 16
LANES = 16
NW = NUM_CORES * NUM_SUBCORES

F = 26
V = 100000
D = 32
B = 16384

SPT = B // NW
CH = 128
NCH = SPT // CH


def _sc_body(tab_hbm, idx_hbm, out_hbm, idx_v, acc_v, buf_v, sem):
    wid = lax.axis_index("s") * NUM_CORES + lax.axis_index("c")
    pltpu.sync_copy(idx_hbm.at[wid], idx_v)

    @pl.loop(0, NCH)
    def _chunk(c):
        pltpu.async_copy(tab_hbm.at[idx_v.at[c]], acc_v, sem).wait()

        @pl.loop(1, F)
        def _field(i):
            pltpu.async_copy(tab_hbm.at[idx_v.at[i * NCH + c]], buf_v, sem).wait()

            @pl.loop(0, CH)
            def _row(r):
                plsc.addupdate(acc_v.at[r, pl.ds(0, LANES)],
                               buf_v[r, pl.ds(0, LANES)])
                plsc.addupdate(acc_v.at[r, pl.ds(LANES, LANES)],
                               buf_v[r, pl.ds(LANES, LANES)])

        base = wid * SPT + c * CH
        pltpu.sync_copy(acc_v, out_hbm.at[pl.ds(base, CH)])


@jax.jit
def kernel(x, tables):
    tab = tables.reshape(F * V, D)
    offs = jnp.arange(F, dtype=jnp.int32) * V
    idx = (x.reshape(NW, NCH, CH, F) + offs).transpose(0, 3, 1, 2)
    idx = idx.reshape(NW, F * NCH, CH)

    k = pl.kernel(
        _sc_body,
        out_type=jax.ShapeDtypeStruct((B, D), jnp.float32),
        compiler_params=pltpu.CompilerParams(use_tc_tiling_on_sc=False),
        mesh=plsc.VectorSubcoreMesh(
            core_axis_name="c", subcore_axis_name="s",
            num_cores=NUM_CORES, num_subcores=NUM_SUBCORES,
        ),
        scratch_types=[
            pltpu.VMEM((F * NCH, CH), jnp.int32),
            pltpu.VMEM((CH, D), jnp.float32),
            pltpu.VMEM((CH, D), jnp.float32),
            pltpu.SemaphoreType.DMA,
        ],
    )
    return k(tab, idx)


# speedup vs baseline: 1.2036x; 1.0803x over previous
import jax
import jax.numpy as jnp
from jax import lax
from jax.experimental import pallas as pl
from jax.experimental.pallas import tpu as pltpu
from jax.experimental.pallas import tpu_sc as plsc

NUM_CORES = 2
NUM_SUBCORES = 16
LANES = 16
NW = NUM_CORES * NUM_SUBCORES

F = 26
V = 100000
D = 32
B = 16384

SPT = B // NW
CH = 128
NCH = SPT // CH
XPT = SPT * F


def _sc_body(tab_hbm, x_hbm, out_hbm, x_v, idx_v, acc_v, sem):
    wid = lax.axis_index("s") * NUM_CORES + lax.axis_index("c")
    pltpu.sync_copy(x_hbm.at[pl.ds(wid * XPT, XPT)], x_v)

    lane_f = lax.iota(jnp.int32, 16) * F

    @pl.loop(0, F * NCH)
    def _build(j):
        i = lax.div(j, NCH)
        c = lax.rem(j, NCH)
        base = c * CH * F + i
        off = i * V

        @pl.loop(0, CH // LANES)
        def _seg(r):
            xi = lane_f + (base + r * LANES * F)
            vals = plsc.load_gather(x_v, [xi])
            idx_v[j, pl.ds(r * LANES, LANES)] = vals + off

    @pl.loop(0, NCH)
    def _init(c):
        pltpu.async_copy(tab_hbm.at[idx_v.at[c]],
                         acc_v.at[pl.ds(c * CH, CH)], sem)

    @pl.loop(0, NCH)
    def _init_drain(c):
        pltpu.make_async_copy(tab_hbm.at[idx_v.at[0]],
                              acc_v.at[pl.ds(0, CH)], sem).wait()

    @pl.loop(NCH, F * NCH)
    def _fire(j):
        c = lax.rem(j, NCH)
        pltpu.async_copy(tab_hbm.at[idx_v.at[j]],
                         acc_v.at[pl.ds(c * CH, CH)], sem, add=True)

    @pl.loop(NCH, F * NCH)
    def _drain(j):
        pltpu.make_async_copy(tab_hbm.at[idx_v.at[0]],
                              acc_v.at[pl.ds(0, CH)], sem).wait()

    pltpu.sync_copy(acc_v, out_hbm.at[pl.ds(wid * SPT, SPT)])


@jax.jit
def kernel(x, tables):
    tab = tables.reshape(F * V, D)
    x_flat = x.reshape(B * F)

    k = pl.kernel(
        _sc_body,
        out_type=jax.ShapeDtypeStruct((B, D), jnp.float32),
        compiler_params=pltpu.CompilerParams(use_tc_tiling_on_sc=False,
                                             needs_layout_passes=False),
        mesh=plsc.VectorSubcoreMesh(
            core_axis_name="c", subcore_axis_name="s",
            num_cores=NUM_CORES, num_subcores=NUM_SUBCORES,
        ),
        scratch_types=[
            pltpu.VMEM((XPT,), jnp.int32),
            pltpu.VMEM((F * NCH, CH), jnp.int32),
            pltpu.VMEM((SPT, D), jnp.float32),
            pltpu.SemaphoreType.DMA,
        ],
    )
    return k(tab, x_flat)


# speedup vs baseline: 1.2061x; 1.0021x over previous
import jax
import jax.numpy as jnp
from jax import lax
from jax.experimental import pallas as pl
from jax.experimental.pallas import tpu as pltpu
from jax.experimental.pallas import tpu_sc as plsc

NUM_CORES = 2
NUM_SUBCORES = 16
LANES = 16
NW = NUM_CORES * NUM_SUBCORES

F = 26
V = 100000
D = 32
B = 16384

SPT = B // NW
CH = 128
NCH = SPT // CH
XPT = SPT * F


def _sc_body(tab_hbm, x_hbm, out_hbm, x_v, idx_v, acc_v, sem):
    wid = lax.axis_index("s") * NUM_CORES + lax.axis_index("c")
    pltpu.sync_copy(x_hbm.at[pl.ds(wid * XPT, XPT)], x_v)

    lane_f = lax.iota(jnp.int32, 16) * F

    def _build_row(j):
        i = lax.div(j, NCH)
        c = lax.rem(j, NCH)
        base = c * CH * F + i
        off = i * V

        @pl.loop(0, CH // LANES)
        def _seg(r):
            xi = lane_f + (base + r * LANES * F)
            vals = plsc.load_gather(x_v, [xi])
            idx_v[j, pl.ds(r * LANES, LANES)] = vals + off

    @pl.loop(0, NCH)
    def _build0(j):
        _build_row(j)

    @pl.loop(0, NCH)
    def _init(c):
        pltpu.async_copy(tab_hbm.at[idx_v.at[c]],
                         acc_v.at[pl.ds(c * CH, CH)], sem)

    @pl.loop(NCH, F * NCH)
    def _build(j):
        _build_row(j)

    @pl.loop(0, NCH)
    def _init_drain(c):
        pltpu.make_async_copy(tab_hbm.at[idx_v.at[0]],
                              acc_v.at[pl.ds(0, CH)], sem).wait()

    @pl.loop(NCH, F * NCH)
    def _fire(j):
        c = lax.rem(j, NCH)
        pltpu.async_copy(tab_hbm.at[idx_v.at[j]],
                         acc_v.at[pl.ds(c * CH, CH)], sem, add=True)

    @pl.loop(NCH, F * NCH)
    def _drain(j):
        pltpu.make_async_copy(tab_hbm.at[idx_v.at[0]],
                              acc_v.at[pl.ds(0, CH)], sem).wait()

    pltpu.sync_copy(acc_v, out_hbm.at[pl.ds(wid * SPT, SPT)])


@jax.jit
def kernel(x, tables):
    tab = tables.reshape(F * V, D)
    x_flat = x.reshape(B * F)

    k = pl.kernel(
        _sc_body,
        out_type=jax.ShapeDtypeStruct((B, D), jnp.float32),
        compiler_params=pltpu.CompilerParams(use_tc_tiling_on_sc=False,
                                             needs_layout_passes=False),
        mesh=plsc.VectorSubcoreMesh(
            core_axis_name="c", subcore_axis_name="s",
            num_cores=NUM_CORES, num_subcores=NUM_SUBCORES,
        ),
        scratch_types=[
            pltpu.VMEM((XPT,), jnp.int32),
            pltpu.VMEM((F * NCH, CH), jnp.int32),
            pltpu.VMEM((SPT, D), jnp.float32),
            pltpu.SemaphoreType.DMA,
        ],
    )
    return k(tab, x_flat)
